# baseline (speedup 1.0000x reference)
import numpy as np
import jax
import jax.numpy as jnp
from jax import lax
from jax.experimental import pallas as pl
from jax.experimental.pallas import tpu as pltpu

_TEMPERATURE = 11.3137
_NUM_SAMPLED = 1024
_EPS = 1e-06
_C = np.float32(_EPS / 4096)

_B = 4
_H = 32
_S = 4096
_SV = 4095
_NEARLY = _NUM_SAMPLED + 1
_ETILE = 128
_NET = 9


def _sig_kernel(q_ref, k_ref, sig_ref, acc32):
    h = pl.program_id(1)
    kb = k_ref[0, 0, :, :]
    qv = q_ref[0, 0, 0, :]
    s = jnp.dot(kb.astype(jnp.bfloat16), qv.astype(jnp.bfloat16),
                preferred_element_type=jnp.float32) / _TEMPERATURE
    e = jnp.exp(s - jnp.max(s))
    den = jnp.sum(e) + np.float32(_EPS)
    a = (e + _C) / den
    acc32[pl.ds(h, 1), :] = a[None, :]

    @pl.when(h == _H - 1)
    def _():
        rows = [acc32[i, :] for i in range(_H)]
        grp = [((rows[t] + rows[8 + t]) + rows[16 + t]) + rows[24 + t]
               for t in range(8)]
        sig = (((grp[0] + grp[4]) + (grp[2] + grp[6]))
               + ((grp[1] + grp[5]) + (grp[3] + grp[7])))
        sig_ref[0, 0, :] = sig


def _rank_mask_kernel(sig_ref, out_ref, acc):
    et = pl.program_id(1)
    row = sig_ref[0, 0, :]
    e0 = et * _ETILE
    ev = sig_ref[0, 0, pl.ds(e0, _ETILE)]
    eidx = e0 + lax.broadcasted_iota(jnp.int32, (_ETILE, 1), 0)
    jidx = lax.broadcasted_iota(jnp.int32, (_ETILE, _SV), 1)
    allv = row[None, :]
    evc = ev[:, None]
    pick = ((allv <= evc) & (jidx < eidx)) | (allv < evc)
    cnt = jnp.dot(
        jnp.where(pick, 1.0, 0.0).astype(jnp.bfloat16),
        jnp.ones((_SV,), jnp.bfloat16),
        preferred_element_type=jnp.float32,
    ).astype(jnp.int32)
    valid = (e0 + lax.iota(jnp.int32, _ETILE)) < _NEARLY
    pos = jnp.where(valid, cnt + 1, 0)
    jfull = lax.broadcasted_iota(jnp.int32, (_ETILE, _S), 1)
    hitf = jnp.dot(
        jnp.ones((_ETILE,), jnp.bfloat16),
        jnp.where(jfull == pos[:, None], 1.0, 0.0).astype(jnp.bfloat16),
        preferred_element_type=jnp.float32,
    )

    @pl.when(et == 0)
    def _():
        acc[0, :] = hitf

    @pl.when(et > 0)
    def _():
        acc[0, :] = acc[0, :] + hitf

    @pl.when(et == _NET - 1)
    def _():
        lane = lax.broadcasted_iota(jnp.int32, (_S,), 0)
        out_ref[0, 0, :] = (lane == 0) | (acc[0, :] > 0.0)


def kernel(q, k, v, token_mask):
    qc = q[:, :, 0, :].reshape(_B, _H, 1, 128)

    sig = pl.pallas_call(
        _sig_kernel,
        grid=(_B, _H),
        in_specs=[
            pl.BlockSpec((1, 1, 1, 128), lambda b, h: (b, h, 0, 0)),
            pl.BlockSpec((1, 1, _S, 128), lambda b, h: (b, h, 0, 0)),
        ],
        out_specs=pl.BlockSpec((1, 1, _S), lambda b, h: (b, 0, 0)),
        out_shape=jax.ShapeDtypeStruct((_B, 1, _S), jnp.float32),
        scratch_shapes=[pltpu.VMEM((_H, _S), jnp.float32)],
    )(qc, k)

    sigs = sig.reshape(_B, _S)[:, 1:]
    sigd = sigs / jnp.sum(sigs, axis=1, keepdims=True)

    mask = pl.pallas_call(
        _rank_mask_kernel,
        grid=(_B, _NET),
        in_specs=[pl.BlockSpec((1, 1, _SV), lambda b, e: (b, 0, 0))],
        out_specs=pl.BlockSpec((1, 1, _S), lambda b, e: (b, 0, 0)),
        out_shape=jax.ShapeDtypeStruct((_B, 1, _S), jnp.bool_),
        scratch_shapes=[pltpu.VMEM((1, _S), jnp.float32)],
    )(sigd.reshape(_B, 1, _SV))
    return mask.reshape(_B, _S)


# speedup vs baseline: 2.5531x; 2.5531x over previous
import numpy as np
import jax
import jax.numpy as jnp
from jax import lax
from jax.experimental import pallas as pl
from jax.experimental.pallas import tpu as pltpu

_TEMPERATURE = 11.3137
_NUM_SAMPLED = 1024
_EPS = 1e-06
_C = np.float32(_EPS / 4096)

_B = 4
_H = 32
_S = 4096
_SV = 4095
_NEARLY = _NUM_SAMPLED + 1
_ETILE = 128
_NET = 9


def _sig_kernel(q_ref, k_ref, sig_ref, acc32):
    h = pl.program_id(1)
    kb = k_ref[0, 0, :, :]
    qv = q_ref[0, 0, 0, :]
    st = lax.dot_general(qv.astype(jnp.bfloat16)[None, :],
                         kb.astype(jnp.bfloat16),
                         (((1,), (1,)), ((), ())),
                         preferred_element_type=jnp.float32)
    s = st[0, :] / _TEMPERATURE
    e = jnp.exp(s - jnp.max(s))
    den = jnp.sum(e) + np.float32(_EPS)
    a = (e + _C) / den
    acc32[pl.ds(h, 1), :] = a[None, :]

    @pl.when(h == _H - 1)
    def _():
        rows = [acc32[i, :] for i in range(_H)]
        grp = [((rows[t] + rows[8 + t]) + rows[16 + t]) + rows[24 + t]
               for t in range(8)]
        sig = (((grp[0] + grp[4]) + (grp[2] + grp[6]))
               + ((grp[1] + grp[5]) + (grp[3] + grp[7])))
        sig_ref[0, 0, :] = sig


def _rank_mask_kernel(sig_ref, out_ref, acc):
    et = pl.program_id(1)
    row = sig_ref[0, 0, :]
    e0 = et * _ETILE
    ev = sig_ref[0, 0, pl.ds(e0, _ETILE)]
    eidx = e0 + lax.broadcasted_iota(jnp.int32, (_ETILE, 1), 0)
    jidx = lax.broadcasted_iota(jnp.int32, (_ETILE, _SV), 1)
    allv = row[None, :]
    evc = ev[:, None]
    pick = ((allv <= evc) & (jidx < eidx)) | (allv < evc)
    cnt = jnp.dot(
        jnp.where(pick, 1.0, 0.0).astype(jnp.bfloat16),
        jnp.ones((_SV,), jnp.bfloat16),
        preferred_element_type=jnp.float32,
    ).astype(jnp.int32)
    valid = (e0 + lax.iota(jnp.int32, _ETILE)) < _NEARLY
    pos = jnp.where(valid, cnt + 1, 0)
    jfull = lax.broadcasted_iota(jnp.int32, (_ETILE, _S), 1)
    hitf = jnp.dot(
        jnp.ones((_ETILE,), jnp.bfloat16),
        jnp.where(jfull == pos[:, None], 1.0, 0.0).astype(jnp.bfloat16),
        preferred_element_type=jnp.float32,
    )

    @pl.when(et == 0)
    def _():
        acc[0, :] = hitf

    @pl.when(et > 0)
    def _():
        acc[0, :] = acc[0, :] + hitf

    @pl.when(et == _NET - 1)
    def _():
        lane = lax.broadcasted_iota(jnp.int32, (_S,), 0)
        out_ref[0, 0, :] = (lane == 0) | (acc[0, :] > 0.0)


def kernel(q, k, v, token_mask):
    qc = q[:, :, 0, :].reshape(_B, _H, 1, 128)

    sig = pl.pallas_call(
        _sig_kernel,
        grid=(_B, _H),
        in_specs=[
            pl.BlockSpec((1, 1, 1, 128), lambda b, h: (b, h, 0, 0)),
            pl.BlockSpec((1, 1, _S, 128), lambda b, h: (b, h, 0, 0)),
        ],
        out_specs=pl.BlockSpec((1, 1, _S), lambda b, h: (b, 0, 0)),
        out_shape=jax.ShapeDtypeStruct((_B, 1, _S), jnp.float32),
        scratch_shapes=[pltpu.VMEM((_H, _S), jnp.float32)],
    )(qc, k)

    sigs = sig.reshape(_B, _S)[:, 1:]
    sigd = sigs / jnp.sum(sigs, axis=1, keepdims=True)

    mask = pl.pallas_call(
        _rank_mask_kernel,
        grid=(_B, _NET),
        in_specs=[pl.BlockSpec((1, 1, _SV), lambda b, e: (b, 0, 0))],
        out_specs=pl.BlockSpec((1, 1, _S), lambda b, e: (b, 0, 0)),
        out_shape=jax.ShapeDtypeStruct((_B, 1, _S), jnp.bool_),
        scratch_shapes=[pltpu.VMEM((1, _S), jnp.float32)],
    )(sigd.reshape(_B, 1, _SV))
    return mask.reshape(_B, _S)


# speedup vs baseline: 3.9813x; 1.5594x over previous
import functools

import jax
import jax.numpy as jnp
from jax import lax
from jax.experimental import pallas as pl
from jax.experimental.pallas import tpu as pltpu
from jax.experimental.pallas import tpu_sc as plsc

_TEMPERATURE = 11.3137
_NUM_SAMPLED = 1024
_EPS = 1e-06

_B = 4
_S = 4096
_SV = 4095
_NEARLY = _NUM_SAMPLED + 1
_ETILE = 128
_NET = 9


def _rank_kernel(sig_ref, rank_ref):
    et = pl.program_id(1)
    row = sig_ref[0, 0, :]
    ev = sig_ref[0, 0, pl.ds(et * _ETILE, _ETILE)]
    eidx = et * _ETILE + lax.broadcasted_iota(jnp.int32, (_ETILE, 1), 0)
    jidx = lax.broadcasted_iota(jnp.int32, (_ETILE, _SV), 1)
    allv = row[None, :]
    evc = ev[:, None]
    lt = allv < evc
    eq_before = (allv == evc) & (jidx < eidx)
    cnt = jnp.sum((lt | eq_before).astype(jnp.int32), axis=1)
    rank_ref[0, 0, :] = cnt


@functools.lru_cache(maxsize=1)
def _make_scatter_kernel():
    mesh = plsc.VectorSubcoreMesh(core_axis_name="c", subcore_axis_name="s")

    @functools.partial(
        pl.kernel,
        mesh=mesh,
        out_type=jax.ShapeDtypeStruct((_B, _S), jnp.float32),
        scratch_types=[
            pltpu.VMEM((_NET * _ETILE,), jnp.int32),
            pltpu.VMEM((_S,), jnp.float32),
        ],
        compiler_params=pltpu.CompilerParams(needs_layout_passes=False),
    )
    def scatter_kernel(ranks_hbm, out_hbm, ranks_v, row_v):
        wid = lax.axis_index("s") * 2 + lax.axis_index("c")

        @pl.when(wid < _B)
        def _():
            pltpu.sync_copy(ranks_hbm.at[wid, 0], ranks_v)
            zeros16 = jnp.zeros((16,), jnp.float32)

            def zero_body(i, _):
                row_v[pl.ds(i * 16, 16)] = zeros16
                return 0

            lax.fori_loop(0, _S // 16, zero_body, 0)

            ones16 = jnp.ones((16,), jnp.float32)
            lane = lax.iota(jnp.int32, 16)

            def scat_body(i, _):
                r16 = ranks_v[pl.ds(i * 16, 16)]
                pos = i * 16 + lane
                valid = pos < _NEARLY
                plsc.store_scatter(row_v, [r16 + 1], ones16, mask=valid)
                return 0

            lax.fori_loop(0, _NET * _ETILE // 16, scat_body, 0)

            head = row_v[pl.ds(0, 16)]
            row_v[pl.ds(0, 16)] = jnp.where(lane == 0, 1.0, head)
            pltpu.sync_copy(row_v, out_hbm.at[wid])

    return scatter_kernel


def kernel(q, k, v, token_mask):
    attn = jnp.matmul(q[..., :1, :], jnp.swapaxes(k, -2, -1)) / _TEMPERATURE
    attn = attn - jnp.max(attn, axis=-1, keepdims=True)
    batch_size, seq_length = token_mask.shape
    attn_mask = token_mask.reshape(batch_size, 1, 1, seq_length)
    attn = jnp.exp(attn) * attn_mask
    attn = (attn + _EPS / seq_length) / (jnp.sum(attn, axis=-1, keepdims=True) + _EPS)
    sig = jnp.sum(attn[:, :, 0], axis=1)
    sig = sig[:, 1:]
    sig = sig / jnp.sum(sig, axis=1, keepdims=True)

    ranks = pl.pallas_call(
        _rank_kernel,
        grid=(_B, _NET),
        in_specs=[pl.BlockSpec((1, 1, _SV), lambda b, e: (b, 0, 0))],
        out_specs=pl.BlockSpec((1, 1, _ETILE), lambda b, e: (b, 0, e)),
        out_shape=jax.ShapeDtypeStruct((_B, 1, _NET * _ETILE), jnp.int32),
    )(sig.reshape(_B, 1, _SV))

    onehot = _make_scatter_kernel()(ranks)
    return onehot != 0.0


# speedup vs baseline: 4.4018x; 1.1056x over previous
import functools

import jax
import jax.numpy as jnp
from jax import lax
from jax.experimental import pallas as pl
from jax.experimental.pallas import tpu as pltpu
from jax.experimental.pallas import tpu_sc as plsc

_TEMPERATURE = 11.3137
_NUM_SAMPLED = 1024
_EPS = 1e-06

_B = 4
_S = 4096
_SV = 4095
_NEARLY = _NUM_SAMPLED + 1
_ETILE = 128
_NET = 9


def _rank_kernel(sig_ref, rank_ref):
    row = sig_ref[0, 0, :]
    jidx = lax.broadcasted_iota(jnp.int32, (_ETILE, _SV), 1)
    allv = row[None, :]
    iota_e = lax.broadcasted_iota(jnp.int32, (_ETILE, 1), 0)
    for et in range(_NET):
        ev = sig_ref[0, 0, pl.ds(et * _ETILE, _ETILE)]
        eidx = et * _ETILE + iota_e
        evc = ev[:, None]
        lt = allv < evc
        eq_before = (allv == evc) & (jidx < eidx)
        cnt = jnp.sum((lt | eq_before).astype(jnp.int32), axis=1)
        rank_ref[0, 0, pl.ds(et * _ETILE, _ETILE)] = cnt


@functools.lru_cache(maxsize=1)
def _make_scatter_kernel():
    mesh = plsc.VectorSubcoreMesh(core_axis_name="c", subcore_axis_name="s")

    @functools.partial(
        pl.kernel,
        mesh=mesh,
        out_type=jax.ShapeDtypeStruct((_B, _S), jnp.float32),
        scratch_types=[
            pltpu.VMEM((_NET * _ETILE,), jnp.int32),
            pltpu.VMEM((_S,), jnp.float32),
        ],
        compiler_params=pltpu.CompilerParams(needs_layout_passes=False),
    )
    def scatter_kernel(ranks_hbm, out_hbm, ranks_v, row_v):
        wid = lax.axis_index("s") * 2 + lax.axis_index("c")

        @pl.when(wid < _B)
        def _():
            pltpu.sync_copy(ranks_hbm.at[wid, 0], ranks_v)
            zeros16 = jnp.zeros((16,), jnp.float32)

            def zero_body(i, _):
                row_v[pl.ds(i * 16, 16)] = zeros16
                return 0

            lax.fori_loop(0, _S // 16, zero_body, 0)

            ones16 = jnp.ones((16,), jnp.float32)
            lane = lax.iota(jnp.int32, 16)

            def scat_body(i, _):
                r16 = ranks_v[pl.ds(i * 16, 16)]
                pos = i * 16 + lane
                valid = pos < _NEARLY
                plsc.store_scatter(row_v, [r16 + 1], ones16, mask=valid)
                return 0

            lax.fori_loop(0, _NET * _ETILE // 16, scat_body, 0)

            head = row_v[pl.ds(0, 16)]
            row_v[pl.ds(0, 16)] = jnp.where(lane == 0, 1.0, head)
            pltpu.sync_copy(row_v, out_hbm.at[wid])

    return scatter_kernel


def kernel(q, k, v, token_mask):
    attn = jnp.matmul(q[..., :1, :], jnp.swapaxes(k, -2, -1)) / _TEMPERATURE
    attn = attn - jnp.max(attn, axis=-1, keepdims=True)
    batch_size, seq_length = token_mask.shape
    attn_mask = token_mask.reshape(batch_size, 1, 1, seq_length)
    attn = jnp.exp(attn) * attn_mask
    attn = (attn + _EPS / seq_length) / (jnp.sum(attn, axis=-1, keepdims=True) + _EPS)
    sig = jnp.sum(attn[:, :, 0], axis=1)
    sig = sig[:, 1:]
    sig = sig / jnp.sum(sig, axis=1, keepdims=True)

    ranks = pl.pallas_call(
        _rank_kernel,
        grid=(_B,),
        in_specs=[pl.BlockSpec((1, 1, _SV), lambda b: (b, 0, 0))],
        out_specs=pl.BlockSpec((1, 1, _NET * _ETILE), lambda b: (b, 0, 0)),
        out_shape=jax.ShapeDtypeStruct((_B, 1, _NET * _ETILE), jnp.int32),
    )(sig.reshape(_B, 1, _SV))

    onehot = _make_scatter_kernel()(ranks)
    return onehot != 0.0
